# baseline (speedup 1.0000x reference)
import jax
import jax.numpy as jnp
from jax import lax
from jax.experimental import pallas as pl
from jax.experimental.pallas import tpu as pltpu
from jax.experimental.pallas import tpu_sc as plsc

_B = 16384
_D = 16
_NC = 2
_NS = 16
_NW = _NC * _NS
_BPW = _B // _NW
_CHUNK = 128
_NE = _D * _BPW


def _bpr_body(uidx_hbm, iidx_hbm, uemb_hbm, iemb_hbm, out_hbm,
              uidx_v, iidx_v, uoff_v, ioff_v, u_v, i_v, out_v, usem, isem):
    wid = lax.axis_index("s") * _NC + lax.axis_index("c")
    base = wid * _BPW

    pltpu.sync_copy(uidx_hbm.at[pl.ds(base, _BPW)], uidx_v)
    pltpu.sync_copy(iidx_hbm.at[pl.ds(base, _BPW)], iidx_v)

    def build(g, carry):
        uv = uidx_v[pl.ds(g * 16, 16)] * 16
        iv = iidx_v[pl.ds(g * 16, 16)] * 16
        for d in range(_D):
            uoff_v[pl.ds(d * _BPW + g * 16, 16)] = uv + d
            ioff_v[pl.ds(d * _BPW + g * 16, 16)] = iv + d
        return carry

    lax.fori_loop(0, _BPW // 16, build, 0)

    copies = []
    for c in range(_NE // _CHUNK):
        sl = pl.ds(c * _CHUNK, _CHUNK)
        copies.append(pltpu.async_copy(
            uemb_hbm.at[uoff_v.at[sl]], u_v.at[sl], usem))
        copies.append(pltpu.async_copy(
            iemb_hbm.at[ioff_v.at[sl]], i_v.at[sl], isem))
    for cc in copies:
        cc.wait()

    def group(g, carry):
        acc = jnp.zeros((16,), jnp.float32)
        for d in range(_D):
            sl = pl.ds(d * _BPW + g * 16, 16)
            acc = acc + u_v[sl] * i_v[sl]
        out_v[pl.ds(g * 16, 16)] = acc
        return carry

    lax.fori_loop(0, _BPW // 16, group, 0)

    pltpu.sync_copy(out_v, out_hbm.at[pl.ds(base, _BPW)])


def kernel(user_idx, item_idx, user_emb, item_emb):
    uemb1 = user_emb.reshape(-1)
    iemb1 = item_emb.reshape(-1)
    mesh = plsc.VectorSubcoreMesh(core_axis_name="c", subcore_axis_name="s")
    f = pl.kernel(
        _bpr_body,
        out_type=jax.ShapeDtypeStruct((_B,), jnp.float32),
        mesh=mesh,
        compiler_params=pltpu.CompilerParams(needs_layout_passes=False),
        scratch_types=[
            pltpu.VMEM((_BPW,), jnp.int32),
            pltpu.VMEM((_BPW,), jnp.int32),
            pltpu.VMEM((_NE,), jnp.int32),
            pltpu.VMEM((_NE,), jnp.int32),
            pltpu.VMEM((_NE,), jnp.float32),
            pltpu.VMEM((_NE,), jnp.float32),
            pltpu.VMEM((_BPW,), jnp.float32),
            pltpu.SemaphoreType.DMA,
            pltpu.SemaphoreType.DMA,
        ],
    )
    return f(user_idx, item_idx, uemb1, iemb1)


# speedup vs baseline: 1.3707x; 1.3707x over previous
import jax
import jax.numpy as jnp
from jax import lax
from jax.experimental import pallas as pl
from jax.experimental.pallas import tpu as pltpu
from jax.experimental.pallas import tpu_sc as plsc

_B = 16384
_D = 16
_NC = 2
_NS = 16
_NW = _NC * _NS
_BPW = _B // _NW
_PASS = 32
_NP = _BPW // _PASS


def _bpr_body(uidx_hbm, iidx_hbm, uemb_hbm, iemb_hbm, out_hbm,
              uidx_v, iidx_v, ublk_v, iblk_v, out_v,
              us0, us1, us2, us3, is0, is1, is2, is3):
    usems = (us0, us1, us2, us3)
    isems = (is0, is1, is2, is3)
    wid = lax.axis_index("s") * _NC + lax.axis_index("c")
    base = wid * _BPW

    pltpu.sync_copy(uidx_hbm.at[pl.ds(base, _BPW)], uidx_v)
    pltpu.sync_copy(iidx_hbm.at[pl.ds(base, _BPW)], iidx_v)

    lanes = lax.iota(jnp.int32, 16)

    for p in range(_NP):
        def fire_body(g, carry):
            uv = uidx_v[pl.ds(p * _PASS + g * 16, 16)]
            iv = iidx_v[pl.ds(p * _PASS + g * 16, 16)]
            for l in range(16):
                us = uv[l]
                is_ = iv[l]
                ublk = pl.multiple_of((us >> 3) * 8, 8)
                iblk = pl.multiple_of((is_ >> 3) * 8, 8)
                i = g * 16 + l
                pltpu.async_copy(uemb_hbm.at[pl.ds(ublk, 8)],
                                 ublk_v.at[pl.ds(i * 8, 8)], usems[l % 4])
                pltpu.async_copy(iemb_hbm.at[pl.ds(iblk, 8)],
                                 iblk_v.at[pl.ds(i * 8, 8)], isems[l % 4])
            return carry

        lax.fori_loop(0, _PASS // 16, fire_body, 0)

        for s in range(4):
            pltpu.make_async_copy(uemb_hbm.at[pl.ds(0, 2 * _PASS)],
                                  ublk_v.at[pl.ds(0, 2 * _PASS)],
                                  usems[s]).wait()
            pltpu.make_async_copy(iemb_hbm.at[pl.ds(0, 2 * _PASS)],
                                  iblk_v.at[pl.ds(0, 2 * _PASS)],
                                  isems[s]).wait()

        def group_body(g, carry):
            rows = g * 16 + lanes
            uv = uidx_v[pl.ds(p * _PASS + g * 16, 16)]
            iv = iidx_v[pl.ds(p * _PASS + g * 16, 16)]
            usub = jnp.bitwise_and(uv, 7)
            isub = jnp.bitwise_and(iv, 7)
            urow = rows * 8 + usub
            irow = rows * 8 + isub
            acc = jnp.zeros((16,), jnp.float32)
            for d in range(_D):
                cols = jnp.full((16,), d, jnp.int32)
                u = plsc.load_gather(ublk_v, [urow, cols])
                v = plsc.load_gather(iblk_v, [irow, cols])
                acc = acc + u * v
            out_v[pl.ds(p * _PASS + g * 16, 16)] = acc
            return carry

        lax.fori_loop(0, _PASS // 16, group_body, 0)

    pltpu.sync_copy(out_v, out_hbm.at[pl.ds(base, _BPW)])


def kernel(user_idx, item_idx, user_emb, item_emb):
    mesh = plsc.VectorSubcoreMesh(core_axis_name="c", subcore_axis_name="s")
    f = pl.kernel(
        _bpr_body,
        out_type=jax.ShapeDtypeStruct((_B,), jnp.float32),
        mesh=mesh,
        compiler_params=pltpu.CompilerParams(needs_layout_passes=False),
        scratch_types=[
            pltpu.VMEM((_BPW,), jnp.int32),
            pltpu.VMEM((_BPW,), jnp.int32),
            pltpu.VMEM((_PASS * 8, _D), jnp.float32),
            pltpu.VMEM((_PASS * 8, _D), jnp.float32),
            pltpu.VMEM((_BPW,), jnp.float32),
            pltpu.SemaphoreType.DMA,
            pltpu.SemaphoreType.DMA,
            pltpu.SemaphoreType.DMA,
            pltpu.SemaphoreType.DMA,
            pltpu.SemaphoreType.DMA,
            pltpu.SemaphoreType.DMA,
            pltpu.SemaphoreType.DMA,
            pltpu.SemaphoreType.DMA,
        ],
    )
    return f(user_idx, item_idx, user_emb, item_emb)


# speedup vs baseline: 1.3756x; 1.0035x over previous
import jax
import jax.numpy as jnp
from jax import lax
from jax.experimental import pallas as pl
from jax.experimental.pallas import tpu as pltpu
from jax.experimental.pallas import tpu_sc as plsc

_B = 16384
_D = 16
_NC = 2
_NS = 16
_NW = _NC * _NS
_BPW = _B // _NW
_PASS = 32
_NP = _BPW // _PASS


def _bpr_body(uidx_hbm, iidx_hbm, uemb_hbm, iemb_hbm, out_hbm,
              uidx_v, iidx_v, ublk_v, iblk_v, out_v,
              usem, isem):
    wid = lax.axis_index("s") * _NC + lax.axis_index("c")
    base = wid * _BPW

    pltpu.sync_copy(uidx_hbm.at[pl.ds(base, _BPW)], uidx_v)
    pltpu.sync_copy(iidx_hbm.at[pl.ds(base, _BPW)], iidx_v)

    lanes = lax.iota(jnp.int32, 16)

    for p in range(_NP):
        def fire_body(g, carry):
            uv = uidx_v[pl.ds(p * _PASS + g * 16, 16)]
            iv = iidx_v[pl.ds(p * _PASS + g * 16, 16)]
            for l in range(16):
                us = uv[l]
                is_ = iv[l]
                ublk = pl.multiple_of((us >> 3) * 8, 8)
                iblk = pl.multiple_of((is_ >> 3) * 8, 8)
                i = g * 16 + l
                pltpu.async_copy(uemb_hbm.at[pl.ds(ublk, 8)],
                                 ublk_v.at[pl.ds(i * 8, 8)], usem)
                pltpu.async_copy(iemb_hbm.at[pl.ds(iblk, 8)],
                                 iblk_v.at[pl.ds(i * 8, 8)], isem)
            return carry

        lax.fori_loop(0, _PASS // 16, fire_body, 0)

        pltpu.make_async_copy(uemb_hbm.at[pl.ds(0, 8 * _PASS)],
                              ublk_v, usem).wait()
        pltpu.make_async_copy(iemb_hbm.at[pl.ds(0, 8 * _PASS)],
                              iblk_v, isem).wait()

        def group_body(g, carry):
            rows = g * 16 + lanes
            uv = uidx_v[pl.ds(p * _PASS + g * 16, 16)]
            iv = iidx_v[pl.ds(p * _PASS + g * 16, 16)]
            usub = jnp.bitwise_and(uv, 7)
            isub = jnp.bitwise_and(iv, 7)
            urow = rows * 8 + usub
            irow = rows * 8 + isub
            acc = jnp.zeros((16,), jnp.float32)
            for d in range(_D):
                cols = jnp.full((16,), d, jnp.int32)
                u = plsc.load_gather(ublk_v, [urow, cols])
                v = plsc.load_gather(iblk_v, [irow, cols])
                acc = acc + u * v
            out_v[pl.ds(p * _PASS + g * 16, 16)] = acc
            return carry

        lax.fori_loop(0, _PASS // 16, group_body, 0)

    pltpu.sync_copy(out_v, out_hbm.at[pl.ds(base, _BPW)])


def kernel(user_idx, item_idx, user_emb, item_emb):
    mesh = plsc.VectorSubcoreMesh(core_axis_name="c", subcore_axis_name="s")
    f = pl.kernel(
        _bpr_body,
        out_type=jax.ShapeDtypeStruct((_B,), jnp.float32),
        mesh=mesh,
        compiler_params=pltpu.CompilerParams(needs_layout_passes=False),
        scratch_types=[
            pltpu.VMEM((_BPW,), jnp.int32),
            pltpu.VMEM((_BPW,), jnp.int32),
            pltpu.VMEM((_PASS * 8, _D), jnp.float32),
            pltpu.VMEM((_PASS * 8, _D), jnp.float32),
            pltpu.VMEM((_BPW,), jnp.float32),
            pltpu.SemaphoreType.DMA,
            pltpu.SemaphoreType.DMA,
        ],
    )
    return f(user_idx, item_idx, user_emb, item_emb)
